# baseline (speedup 1.0000x reference)
import functools

import jax
import jax.numpy as jnp
from jax import lax
from jax.experimental import pallas as pl
from jax.experimental.pallas import tpu as pltpu
from jax.experimental.pallas import tpu_sc as plsc

N = 10000
E = 320000
F = 128
NG = 64

NC = 2
NS = 16
NW = NC * NS
CH = 128
NPAD = 10240
RPT = NPAD // NS
NB = 4
G = 2
K0 = 152
K1 = 8
KMAX = max(K0, K1)
TCH = NS * (K0 + K1)
E_PAD = TCH * CH
DCPT = TCH // NW


def _make_sc_scatter(H):
    mesh = plsc.VectorSubcoreMesh(core_axis_name="c", subcore_axis_name="s")

    @functools.partial(
        pl.kernel,
        out_type=jax.ShapeDtypeStruct((NC, NPAD, H), jnp.float32),
        mesh=mesh,
        scratch_types=[
            pltpu.VMEM((KMAX, CH), jnp.int32),
            pltpu.VMEM((KMAX, CH), jnp.int32),
            [pltpu.VMEM((CH, H), jnp.float32)] * NB,
            pltpu.VMEM_SHARED((NPAD, H), jnp.float32),
            [pltpu.SemaphoreType.DMA] * NB,
            [pltpu.SemaphoreType.DMA] * NB,
        ],
        compiler_params=pltpu.CompilerParams(use_tc_tiling_on_sc=False),
    )
    def k(y_hbm, src_hbm, dst_hbm, out_hbm, src_i, dst_i, rows, acc_sh,
          gsems, ssems):
        cid = lax.axis_index("c")
        sid = lax.axis_index("s")

        zero16 = jnp.zeros((16,), jnp.float32)

        def zrow(r, carry):
            for j in range(H // 16):
                rows[0][r, pl.ds(j * 16, 16)] = zero16
            return carry

        lax.fori_loop(0, CH, zrow, 0)
        for t in range(RPT // CH):
            pltpu.sync_copy(rows[0], acc_sh.at[pl.ds(sid * RPT + t * CH, CH)])
        plsc.subcore_barrier()

        def run(base_row, K):
            if K == 0:
                return
            pltpu.sync_copy(src_hbm.at[pl.ds(base_row, K)],
                            src_i.at[pl.ds(0, K)])
            pltpu.sync_copy(dst_hbm.at[pl.ds(base_row, K)],
                            dst_i.at[pl.ds(0, K)])
            for b in range(G):
                pltpu.async_copy(y_hbm.at[src_i.at[b]], rows[b], gsems[b])

            def body(o, carry):
                for b in range(NB):
                    g = o * NB + b
                    gp = g + G
                    bp = (b + G) % NB
                    pltpu.make_async_copy(y_hbm.at[src_i.at[g]], rows[b],
                                          gsems[b]).wait()
                    pltpu.async_copy(rows[b], acc_sh.at[dst_i.at[g]],
                                     ssems[b], add=True)

                    @pl.when(jnp.logical_and(gp < K, gp >= NB))
                    def _():
                        pltpu.make_async_copy(rows[bp],
                                              acc_sh.at[dst_i.at[gp - NB]],
                                              ssems[bp]).wait()

                    @pl.when(gp < K)
                    def _():
                        pltpu.async_copy(y_hbm.at[src_i.at[gp]], rows[bp],
                                         gsems[bp])
                return carry

            lax.fori_loop(0, K // NB, body, 0)
            for b in range(NB):
                pltpu.make_async_copy(rows[b], acc_sh.at[dst_i.at[0]],
                                      ssems[b]).wait()

        @pl.when(cid == 0)
        def _():
            run(sid * K0, K0)

        @pl.when(cid == 1)
        def _():
            run(NS * K0 + sid * K1, K1)

        plsc.subcore_barrier()
        pltpu.sync_copy(
            acc_sh.at[pl.ds(sid * RPT, RPT)],
            out_hbm.at[cid, pl.ds(sid * RPT, RPT)],
        )

    return k


def _sc_degree():
    mesh = plsc.VectorSubcoreMesh(core_axis_name="c", subcore_axis_name="s")

    @functools.partial(
        pl.kernel,
        out_type=jax.ShapeDtypeStruct((NC, NPAD, 16), jnp.float32),
        mesh=mesh,
        scratch_types=[
            pltpu.VMEM((DCPT, CH), jnp.int32),
            pltpu.VMEM((CH, 16), jnp.float32),
            pltpu.VMEM_SHARED((NPAD, 16), jnp.float32),
        ],
        compiler_params=pltpu.CompilerParams(use_tc_tiling_on_sc=False),
    )
    def k(dst_hbm, out_hbm, dst_i, ones_v, acc_sh):
        cid = lax.axis_index("c")
        sid = lax.axis_index("s")
        base_row = (cid * NS + sid) * DCPT

        pltpu.sync_copy(dst_hbm.at[pl.ds(base_row, DCPT)], dst_i)

        zero16 = jnp.zeros((16,), jnp.float32)
        one16 = jnp.ones((16,), jnp.float32)

        def frow(r, carry):
            ones_v[r, :] = zero16
            return carry

        lax.fori_loop(0, CH, frow, 0)
        for t in range(RPT // CH):
            pltpu.sync_copy(ones_v, acc_sh.at[pl.ds(sid * RPT + t * CH, CH)])

        def orow(r, carry):
            ones_v[r, :] = one16
            return carry

        lax.fori_loop(0, CH, orow, 0)
        plsc.subcore_barrier()

        def body(g, carry):
            pltpu.sync_copy(ones_v, acc_sh.at[dst_i.at[g]], add=True)
            return carry

        lax.fori_loop(0, DCPT, body, 0)
        plsc.subcore_barrier()
        pltpu.sync_copy(
            acc_sh.at[pl.ds(sid * RPT, RPT)],
            out_hbm.at[cid, pl.ds(sid * RPT, RPT)],
        )

    return k


_sc_scatter64 = _make_sc_scatter(64)
_sc_scatter32 = _make_sc_scatter(32)
_sc_deg = _sc_degree()

BR = 512


def _prep1_body(x_ref, w_ref, degp_ref, y_ref, dinv_ref):
    i = pl.program_id(0)
    deg = degp_ref[0, :, 0:1] + degp_ref[1, :, 0:1] + 1.0
    rows = lax.broadcasted_iota(jnp.int32, (BR, 1), 0) + i * BR
    dinv = jnp.where(rows < N, lax.rsqrt(deg), 0.0)
    dinv_ref[...] = dinv
    y_ref[...] = dinv * jnp.dot(x_ref[...], w_ref[...],
                                preferred_element_type=jnp.float32)


def _comb_body(sp_ref, y_ref, dinv_ref, b_ref, w_ref, ynext_ref):
    dinv = dinv_ref[...]
    h = jax.nn.relu(dinv * (sp_ref[0] + sp_ref[1] + y_ref[...]) + b_ref[...])
    ynext_ref[...] = dinv * jnp.dot(h, w_ref[...],
                                    preferred_element_type=jnp.float32)


def _final_body(sp_ref, y_ref, dinv_ref, b_ref, batch_ref, wf1_ref, bf1_ref,
                wf2_ref, bf2_ref, out_ref):
    dinv = dinv_ref[...]
    h = jax.nn.relu(dinv * (sp_ref[0] + sp_ref[1] + y_ref[...]) + b_ref[...])
    gids = lax.broadcasted_iota(jnp.int32, (NG, NPAD), 0)
    m = (batch_ref[...] == gids).astype(jnp.float32)
    sums = jnp.dot(m, h, preferred_element_type=jnp.float32)
    counts = jnp.sum(m, axis=1, keepdims=True)
    pooled = sums / jnp.maximum(counts, 1.0)
    h2 = jax.nn.relu(jnp.dot(pooled, wf1_ref[...],
                             preferred_element_type=jnp.float32) + bf1_ref[...])
    out_ref[...] = jnp.dot(h2, wf2_ref[...],
                           preferred_element_type=jnp.float32) + bf2_ref[...]


def _prep1(x_pad, W1, degp):
    grid = (NPAD // BR,)
    return pl.pallas_call(
        _prep1_body,
        grid=grid,
        in_specs=[
            pl.BlockSpec((BR, F), lambda i: (i, 0)),
            pl.BlockSpec((F, 64), lambda i: (0, 0)),
            pl.BlockSpec((NC, BR, 16), lambda i: (0, i, 0)),
        ],
        out_specs=[
            pl.BlockSpec((BR, 64), lambda i: (i, 0)),
            pl.BlockSpec((BR, 1), lambda i: (i, 0)),
        ],
        out_shape=[
            jax.ShapeDtypeStruct((NPAD, 64), jnp.float32),
            jax.ShapeDtypeStruct((NPAD, 1), jnp.float32),
        ],
    )(x_pad, W1, degp)


def _comb(sp, y, dinv, b, W, Hin, Hout):
    grid = (NPAD // BR,)
    return pl.pallas_call(
        _comb_body,
        grid=grid,
        in_specs=[
            pl.BlockSpec((NC, BR, Hin), lambda i: (0, i, 0)),
            pl.BlockSpec((BR, Hin), lambda i: (i, 0)),
            pl.BlockSpec((BR, 1), lambda i: (i, 0)),
            pl.BlockSpec((1, Hin), lambda i: (0, 0)),
            pl.BlockSpec((Hin, Hout), lambda i: (0, 0)),
        ],
        out_specs=pl.BlockSpec((BR, Hout), lambda i: (i, 0)),
        out_shape=jax.ShapeDtypeStruct((NPAD, Hout), jnp.float32),
    )(sp, y, dinv, b, W)


def _final(sp, y, dinv, b3, batch_row, Wf1, bf1, Wf2, bf2):
    return pl.pallas_call(
        _final_body,
        out_shape=jax.ShapeDtypeStruct((NG, 10), jnp.float32),
    )(sp, y, dinv, b3, batch_row, Wf1, bf1, Wf2, bf2)


def kernel(x, edge_index, batch, W1, b1, W2, b2, W3, b3, Wf1, bf1, Wf2, bf2):
    src = jnp.concatenate([edge_index[0],
                           jnp.full((E_PAD - E,), N, jnp.int32)])
    dst = jnp.concatenate([edge_index[1],
                           jnp.full((E_PAD - E,), N, jnp.int32)])
    src = src.reshape(TCH, CH)
    dst = dst.reshape(TCH, CH)
    x_pad = jnp.zeros((NPAD, F), jnp.float32).at[:N].set(x)
    batch_row = jnp.full((1, NPAD), NG, jnp.int32).at[0, :N].set(batch)

    degp = _sc_deg(dst)

    y1, dinv = _prep1(x_pad, W1, degp)
    s1 = _sc_scatter64(y1, src, dst)
    y2 = _comb(s1, y1, dinv, b1.reshape(1, 64), W2, 64, 64)
    s2 = _sc_scatter64(y2, src, dst)
    y3 = _comb(s2, y2, dinv, b2.reshape(1, 64), W3, 64, 32)
    s3 = _sc_scatter32(y3, src, dst)
    return _final(s3, y3, dinv, b3.reshape(1, 32), batch_row,
                  Wf1, bf1.reshape(1, 32), Wf2, bf2.reshape(1, 10))


# speedup vs baseline: 1.7663x; 1.7663x over previous
import functools

import jax
import jax.numpy as jnp
from jax import lax
from jax.experimental import pallas as pl
from jax.experimental.pallas import tpu as pltpu
from jax.experimental.pallas import tpu_sc as plsc

N = 10000
E = 320000
F = 128
NG = 64

NC = 2
NS = 16
NW = NC * NS
CH = 128
NPAD = 10240
RPT = NPAD // NS
NB = 2

K0 = 80
K1 = 80
KMAX = max(K0, K1)
TCH = NS * (K0 + K1)
E_PAD = TCH * CH
DCPT = TCH // NW


def _make_sc_scatter(H):
    mesh = plsc.VectorSubcoreMesh(core_axis_name="c", subcore_axis_name="s")

    @functools.partial(
        pl.kernel,
        out_type=jax.ShapeDtypeStruct((NC, NPAD, H), jnp.float32),
        mesh=mesh,
        scratch_types=[
            pltpu.VMEM((KMAX, CH), jnp.int32),
            pltpu.VMEM((KMAX, CH), jnp.int32),
            [pltpu.VMEM((CH, H), jnp.float32)] * NB,
            pltpu.VMEM_SHARED((NPAD, H), jnp.float32),
            pltpu.VMEM_SHARED((NPAD, H), jnp.float32),
            [pltpu.SemaphoreType.DMA] * NB,
        ],
        compiler_params=pltpu.CompilerParams(use_tc_tiling_on_sc=False),
    )
    def k(y_hbm, src_hbm, dst_hbm, out_hbm, src_i, dst_i, rows, acc_sh,
          y_sh, gsems):
        cid = lax.axis_index("c")
        sid = lax.axis_index("s")

        pltpu.sync_copy(y_hbm.at[pl.ds(sid * RPT, RPT)],
                        y_sh.at[pl.ds(sid * RPT, RPT)])

        zero16 = jnp.zeros((16,), jnp.float32)

        def zrow(r, carry):
            for j in range(H // 16):
                rows[0][r, pl.ds(j * 16, 16)] = zero16
            return carry

        lax.fori_loop(0, CH, zrow, 0)
        for t in range(RPT // CH):
            pltpu.sync_copy(rows[0], acc_sh.at[pl.ds(sid * RPT + t * CH, CH)])
        plsc.subcore_barrier()

        def run(base_row, K):
            if K == 0:
                return
            pltpu.sync_copy(src_hbm.at[pl.ds(base_row, K)],
                            src_i.at[pl.ds(0, K)])
            pltpu.sync_copy(dst_hbm.at[pl.ds(base_row, K)],
                            dst_i.at[pl.ds(0, K)])
            for b in range(NB):
                pltpu.async_copy(y_sh.at[src_i.at[b]], rows[b], gsems[b])

            def body(o, carry):
                for b in range(NB):
                    g = o * NB + b
                    pltpu.make_async_copy(y_sh.at[src_i.at[g]], rows[b],
                                          gsems[b]).wait()
                    pltpu.sync_copy(rows[b], acc_sh.at[dst_i.at[g]], add=True)

                    @pl.when(g + NB < K)
                    def _():
                        pltpu.async_copy(y_sh.at[src_i.at[g + NB]], rows[b],
                                         gsems[b])
                return carry

            lax.fori_loop(0, K // NB, body, 0)

        @pl.when(cid == 0)
        def _():
            run(sid * K0, K0)

        @pl.when(cid == 1)
        def _():
            run(NS * K0 + sid * K1, K1)

        plsc.subcore_barrier()
        pltpu.sync_copy(
            acc_sh.at[pl.ds(sid * RPT, RPT)],
            out_hbm.at[cid, pl.ds(sid * RPT, RPT)],
        )

    return k


def _sc_degree():
    mesh = plsc.VectorSubcoreMesh(core_axis_name="c", subcore_axis_name="s")

    @functools.partial(
        pl.kernel,
        out_type=jax.ShapeDtypeStruct((NC, NPAD, 16), jnp.float32),
        mesh=mesh,
        scratch_types=[
            pltpu.VMEM((DCPT, CH), jnp.int32),
            pltpu.VMEM((CH, 16), jnp.float32),
            pltpu.VMEM_SHARED((NPAD, 16), jnp.float32),
        ],
        compiler_params=pltpu.CompilerParams(use_tc_tiling_on_sc=False),
    )
    def k(dst_hbm, out_hbm, dst_i, ones_v, acc_sh):
        cid = lax.axis_index("c")
        sid = lax.axis_index("s")
        base_row = (cid * NS + sid) * DCPT

        pltpu.sync_copy(dst_hbm.at[pl.ds(base_row, DCPT)], dst_i)

        zero16 = jnp.zeros((16,), jnp.float32)
        one16 = jnp.ones((16,), jnp.float32)

        def frow(r, carry):
            ones_v[r, :] = zero16
            return carry

        lax.fori_loop(0, CH, frow, 0)
        for t in range(RPT // CH):
            pltpu.sync_copy(ones_v, acc_sh.at[pl.ds(sid * RPT + t * CH, CH)])

        def orow(r, carry):
            ones_v[r, :] = one16
            return carry

        lax.fori_loop(0, CH, orow, 0)
        plsc.subcore_barrier()

        def body(g, carry):
            pltpu.sync_copy(ones_v, acc_sh.at[dst_i.at[g]], add=True)
            return carry

        lax.fori_loop(0, DCPT, body, 0)
        plsc.subcore_barrier()
        pltpu.sync_copy(
            acc_sh.at[pl.ds(sid * RPT, RPT)],
            out_hbm.at[cid, pl.ds(sid * RPT, RPT)],
        )

    return k


_sc_scatter64 = _make_sc_scatter(64)
_sc_scatter32 = _make_sc_scatter(32)
_sc_deg = _sc_degree()

BR = 512


def _prep1_body(x_ref, w_ref, degp_ref, y_ref, dinv_ref):
    i = pl.program_id(0)
    deg = degp_ref[0, :, 0:1] + degp_ref[1, :, 0:1] + 1.0
    rows = lax.broadcasted_iota(jnp.int32, (BR, 1), 0) + i * BR
    dinv = jnp.where(rows < N, lax.rsqrt(deg), 0.0)
    dinv_ref[...] = dinv
    y_ref[...] = dinv * jnp.dot(x_ref[...], w_ref[...],
                                preferred_element_type=jnp.float32)


def _comb_body(sp_ref, y_ref, dinv_ref, b_ref, w_ref, ynext_ref):
    dinv = dinv_ref[...]
    h = jax.nn.relu(dinv * (sp_ref[0] + sp_ref[1] + y_ref[...]) + b_ref[...])
    ynext_ref[...] = dinv * jnp.dot(h, w_ref[...],
                                    preferred_element_type=jnp.float32)


def _final_body(sp_ref, y_ref, dinv_ref, b_ref, batch_ref, wf1_ref, bf1_ref,
                wf2_ref, bf2_ref, out_ref):
    dinv = dinv_ref[...]
    h = jax.nn.relu(dinv * (sp_ref[0] + sp_ref[1] + y_ref[...]) + b_ref[...])
    gids = lax.broadcasted_iota(jnp.int32, (NG, NPAD), 0)
    m = (batch_ref[...] == gids).astype(jnp.float32)
    sums = jnp.dot(m, h, preferred_element_type=jnp.float32)
    counts = jnp.sum(m, axis=1, keepdims=True)
    pooled = sums / jnp.maximum(counts, 1.0)
    h2 = jax.nn.relu(jnp.dot(pooled, wf1_ref[...],
                             preferred_element_type=jnp.float32) + bf1_ref[...])
    out_ref[...] = jnp.dot(h2, wf2_ref[...],
                           preferred_element_type=jnp.float32) + bf2_ref[...]


def _prep1(x_pad, W1, degp):
    grid = (NPAD // BR,)
    return pl.pallas_call(
        _prep1_body,
        grid=grid,
        in_specs=[
            pl.BlockSpec((BR, F), lambda i: (i, 0)),
            pl.BlockSpec((F, 64), lambda i: (0, 0)),
            pl.BlockSpec((NC, BR, 16), lambda i: (0, i, 0)),
        ],
        out_specs=[
            pl.BlockSpec((BR, 64), lambda i: (i, 0)),
            pl.BlockSpec((BR, 1), lambda i: (i, 0)),
        ],
        out_shape=[
            jax.ShapeDtypeStruct((NPAD, 64), jnp.float32),
            jax.ShapeDtypeStruct((NPAD, 1), jnp.float32),
        ],
    )(x_pad, W1, degp)


def _comb(sp, y, dinv, b, W, Hin, Hout):
    grid = (NPAD // BR,)
    return pl.pallas_call(
        _comb_body,
        grid=grid,
        in_specs=[
            pl.BlockSpec((NC, BR, Hin), lambda i: (0, i, 0)),
            pl.BlockSpec((BR, Hin), lambda i: (i, 0)),
            pl.BlockSpec((BR, 1), lambda i: (i, 0)),
            pl.BlockSpec((1, Hin), lambda i: (0, 0)),
            pl.BlockSpec((Hin, Hout), lambda i: (0, 0)),
        ],
        out_specs=pl.BlockSpec((BR, Hout), lambda i: (i, 0)),
        out_shape=jax.ShapeDtypeStruct((NPAD, Hout), jnp.float32),
    )(sp, y, dinv, b, W)


def _final(sp, y, dinv, b3, batch_row, Wf1, bf1, Wf2, bf2):
    return pl.pallas_call(
        _final_body,
        out_shape=jax.ShapeDtypeStruct((NG, 10), jnp.float32),
    )(sp, y, dinv, b3, batch_row, Wf1, bf1, Wf2, bf2)


def kernel(x, edge_index, batch, W1, b1, W2, b2, W3, b3, Wf1, bf1, Wf2, bf2):
    src = jnp.concatenate([edge_index[0],
                           jnp.full((E_PAD - E,), N, jnp.int32)])
    dst = jnp.concatenate([edge_index[1],
                           jnp.full((E_PAD - E,), N, jnp.int32)])
    src = src.reshape(TCH, CH)
    dst = dst.reshape(TCH, CH)
    x_pad = jnp.zeros((NPAD, F), jnp.float32).at[:N].set(x)
    batch_row = jnp.full((1, NPAD), NG, jnp.int32).at[0, :N].set(batch)

    degp = _sc_deg(dst)

    y1, dinv = _prep1(x_pad, W1, degp)
    s1 = _sc_scatter64(y1, src, dst)
    y2 = _comb(s1, y1, dinv, b1.reshape(1, 64), W2, 64, 64)
    s2 = _sc_scatter64(y2, src, dst)
    y3 = _comb(s2, y2, dinv, b2.reshape(1, 64), W3, 64, 32)
    s3 = _sc_scatter32(y3, src, dst)
    return _final(s3, y3, dinv, b3.reshape(1, 32), batch_row,
                  Wf1, bf1.reshape(1, 32), Wf2, bf2.reshape(1, 10))


# speedup vs baseline: 1.8539x; 1.0496x over previous
import functools

import jax
import jax.numpy as jnp
from jax import lax
from jax.experimental import pallas as pl
from jax.experimental.pallas import tpu as pltpu
from jax.experimental.pallas import tpu_sc as plsc

N = 10000
E = 320000
F = 128
NG = 64

NC = 2
NS = 16
NW = NC * NS
CH = 128
NPAD = 10240
RPT = NPAD // NS
NB = 2

K0 = 80
K1 = 80
KMAX = max(K0, K1)
TCH = NS * (K0 + K1)
E_PAD = TCH * CH
DCPT = TCH // NW


def _make_sc_scatter(H):
    mesh = plsc.VectorSubcoreMesh(core_axis_name="c", subcore_axis_name="s")

    @functools.partial(
        pl.kernel,
        out_type=jax.ShapeDtypeStruct((NC, NPAD, H), jnp.float32),
        mesh=mesh,
        scratch_types=[
            pltpu.VMEM((KMAX, CH), jnp.int32),
            pltpu.VMEM((KMAX, CH), jnp.int32),
            [pltpu.VMEM((CH, H), jnp.float32)] * NB,
            pltpu.VMEM_SHARED((NPAD, H), jnp.float32),
            pltpu.VMEM_SHARED((NPAD, H), jnp.float32),
            [pltpu.SemaphoreType.DMA] * NB,
        ],
        compiler_params=pltpu.CompilerParams(use_tc_tiling_on_sc=False),
    )
    def k(y_hbm, src_hbm, dst_hbm, out_hbm, src_i, dst_i, rows, acc_sh,
          y_sh, gsems):
        cid = lax.axis_index("c")
        sid = lax.axis_index("s")

        pltpu.sync_copy(y_hbm.at[pl.ds(sid * RPT, RPT)],
                        y_sh.at[pl.ds(sid * RPT, RPT)])

        zero16 = jnp.zeros((16,), jnp.float32)

        def zrow(r, carry):
            for j in range(H // 16):
                rows[0][r, pl.ds(j * 16, 16)] = zero16
            return carry

        lax.fori_loop(0, CH, zrow, 0)
        for t in range(RPT // CH):
            pltpu.sync_copy(rows[0], acc_sh.at[pl.ds(sid * RPT + t * CH, CH)])
        plsc.subcore_barrier()

        def run(base_row, K):
            if K == 0:
                return
            pltpu.sync_copy(src_hbm.at[pl.ds(base_row, K)],
                            src_i.at[pl.ds(0, K)])
            pltpu.sync_copy(dst_hbm.at[pl.ds(base_row, K)],
                            dst_i.at[pl.ds(0, K)])
            for b in range(NB):
                pltpu.async_copy(y_sh.at[src_i.at[b]], rows[b], gsems[b])

            def body(o, carry):
                for b in range(NB):
                    g = o * NB + b
                    pltpu.make_async_copy(y_sh.at[src_i.at[g]], rows[b],
                                          gsems[b]).wait()
                    pltpu.sync_copy(rows[b], acc_sh.at[dst_i.at[g]], add=True)

                    @pl.when(g + NB < K)
                    def _():
                        pltpu.async_copy(y_sh.at[src_i.at[g + NB]], rows[b],
                                         gsems[b])
                return carry

            lax.fori_loop(0, K // NB, body, 0)

        @pl.when(cid == 0)
        def _():
            run(sid * K0, K0)

        @pl.when(cid == 1)
        def _():
            run(NS * K0 + sid * K1, K1)

        plsc.subcore_barrier()
        pltpu.sync_copy(
            acc_sh.at[pl.ds(sid * RPT, RPT)],
            out_hbm.at[cid, pl.ds(sid * RPT, RPT)],
        )

    return k


def _sc_degree():
    mesh = plsc.VectorSubcoreMesh(core_axis_name="c", subcore_axis_name="s")

    @functools.partial(
        pl.kernel,
        out_type=jax.ShapeDtypeStruct((NC, NPAD, 16), jnp.float32),
        mesh=mesh,
        scratch_types=[
            pltpu.VMEM((DCPT, CH), jnp.int32),
            pltpu.VMEM((CH, 16), jnp.float32),
            pltpu.VMEM_SHARED((NPAD, 16), jnp.float32),
        ],
        compiler_params=pltpu.CompilerParams(use_tc_tiling_on_sc=False),
    )
    def k(dst_hbm, out_hbm, dst_i, ones_v, acc_sh):
        cid = lax.axis_index("c")
        sid = lax.axis_index("s")
        base_row = (cid * NS + sid) * DCPT

        pltpu.sync_copy(dst_hbm.at[pl.ds(base_row, DCPT)], dst_i)

        zero16 = jnp.zeros((16,), jnp.float32)
        one16 = jnp.ones((16,), jnp.float32)

        def frow(r, carry):
            ones_v[r, :] = zero16
            return carry

        lax.fori_loop(0, CH, frow, 0)
        for t in range(RPT // CH):
            pltpu.sync_copy(ones_v, acc_sh.at[pl.ds(sid * RPT + t * CH, CH)])

        def orow(r, carry):
            ones_v[r, :] = one16
            return carry

        lax.fori_loop(0, CH, orow, 0)
        plsc.subcore_barrier()

        def body(g, carry):
            pltpu.sync_copy(ones_v, acc_sh.at[dst_i.at[g]], add=True)
            return carry

        lax.fori_loop(0, DCPT, body, 0)
        plsc.subcore_barrier()
        pltpu.sync_copy(
            acc_sh.at[pl.ds(sid * RPT, RPT)],
            out_hbm.at[cid, pl.ds(sid * RPT, RPT)],
        )

    return k


_sc_scatter64 = _make_sc_scatter(64)
_sc_scatter32 = _make_sc_scatter(32)
_sc_deg = _sc_degree()

BR = 1024


def _prep1_body(x_ref, w_ref, degp_ref, y_ref, dinv_ref):
    i = pl.program_id(0)
    deg = degp_ref[0, :, 0:1] + degp_ref[1, :, 0:1] + 1.0
    rows = lax.broadcasted_iota(jnp.int32, (BR, 1), 0) + i * BR
    dinv = jnp.where(rows < N, lax.rsqrt(deg), 0.0)
    dinv_ref[...] = dinv
    y_ref[...] = dinv * jnp.dot(x_ref[...], w_ref[...],
                                preferred_element_type=jnp.float32)


def _comb_body(sp_ref, y_ref, dinv_ref, b_ref, w_ref, ynext_ref):
    dinv = dinv_ref[...]
    h = jax.nn.relu(dinv * (sp_ref[0] + sp_ref[1] + y_ref[...]) + b_ref[...])
    ynext_ref[...] = dinv * jnp.dot(h, w_ref[...],
                                    preferred_element_type=jnp.float32)


def _final_body(sp_ref, y_ref, dinv_ref, b_ref, batch_ref, wf1_ref, bf1_ref,
                wf2_ref, bf2_ref, out_ref):
    dinv = dinv_ref[...]
    h = jax.nn.relu(dinv * (sp_ref[0] + sp_ref[1] + y_ref[...]) + b_ref[...])
    gids = lax.broadcasted_iota(jnp.int32, (NG, NPAD), 0)
    m = (batch_ref[...] == gids).astype(jnp.float32)
    sums = jnp.dot(m, h, preferred_element_type=jnp.float32)
    counts = jnp.sum(m, axis=1, keepdims=True)
    pooled = sums / jnp.maximum(counts, 1.0)
    h2 = jax.nn.relu(jnp.dot(pooled, wf1_ref[...],
                             preferred_element_type=jnp.float32) + bf1_ref[...])
    out_ref[...] = jnp.dot(h2, wf2_ref[...],
                           preferred_element_type=jnp.float32) + bf2_ref[...]


def _prep1(x_pad, W1, degp):
    grid = (NPAD // BR,)
    return pl.pallas_call(
        _prep1_body,
        grid=grid,
        in_specs=[
            pl.BlockSpec((BR, F), lambda i: (i, 0)),
            pl.BlockSpec((F, 64), lambda i: (0, 0)),
            pl.BlockSpec((NC, BR, 16), lambda i: (0, i, 0)),
        ],
        out_specs=[
            pl.BlockSpec((BR, 64), lambda i: (i, 0)),
            pl.BlockSpec((BR, 1), lambda i: (i, 0)),
        ],
        out_shape=[
            jax.ShapeDtypeStruct((NPAD, 64), jnp.float32),
            jax.ShapeDtypeStruct((NPAD, 1), jnp.float32),
        ],
    )(x_pad, W1, degp)


def _comb(sp, y, dinv, b, W, Hin, Hout):
    grid = (NPAD // BR,)
    return pl.pallas_call(
        _comb_body,
        grid=grid,
        in_specs=[
            pl.BlockSpec((NC, BR, Hin), lambda i: (0, i, 0)),
            pl.BlockSpec((BR, Hin), lambda i: (i, 0)),
            pl.BlockSpec((BR, 1), lambda i: (i, 0)),
            pl.BlockSpec((1, Hin), lambda i: (0, 0)),
            pl.BlockSpec((Hin, Hout), lambda i: (0, 0)),
        ],
        out_specs=pl.BlockSpec((BR, Hout), lambda i: (i, 0)),
        out_shape=jax.ShapeDtypeStruct((NPAD, Hout), jnp.float32),
    )(sp, y, dinv, b, W)


def _final(sp, y, dinv, b3, batch_row, Wf1, bf1, Wf2, bf2):
    return pl.pallas_call(
        _final_body,
        out_shape=jax.ShapeDtypeStruct((NG, 10), jnp.float32),
    )(sp, y, dinv, b3, batch_row, Wf1, bf1, Wf2, bf2)


def kernel(x, edge_index, batch, W1, b1, W2, b2, W3, b3, Wf1, bf1, Wf2, bf2):
    src = jnp.concatenate([edge_index[0],
                           jnp.full((E_PAD - E,), N, jnp.int32)])
    dst = jnp.concatenate([edge_index[1],
                           jnp.full((E_PAD - E,), N, jnp.int32)])
    src = src.reshape(TCH, CH)
    dst = dst.reshape(TCH, CH)
    x_pad = jnp.zeros((NPAD, F), jnp.float32).at[:N].set(x)
    batch_row = jnp.full((1, NPAD), NG, jnp.int32).at[0, :N].set(batch)

    degp = _sc_deg(dst)

    y1, dinv = _prep1(x_pad, W1, degp)
    s1 = _sc_scatter64(y1, src, dst)
    y2 = _comb(s1, y1, dinv, b1.reshape(1, 64), W2, 64, 64)
    s2 = _sc_scatter64(y2, src, dst)
    y3 = _comb(s2, y2, dinv, b2.reshape(1, 64), W3, 64, 32)
    s3 = _sc_scatter32(y3, src, dst)
    return _final(s3, y3, dinv, b3.reshape(1, 32), batch_row,
                  Wf1, bf1.reshape(1, 32), Wf2, bf2.reshape(1, 10))


# speedup vs baseline: 1.8880x; 1.0184x over previous
import functools

import jax
import jax.numpy as jnp
from jax import lax
from jax.experimental import pallas as pl
from jax.experimental.pallas import tpu as pltpu
from jax.experimental.pallas import tpu_sc as plsc

N = 10000
E = 320000
F = 128
NG = 64

NC = 2
NS = 16
NW = NC * NS
CH = 128
NPAD = 10240
RPT = NPAD // NS
NB = 2

K0 = 80
K1 = 80
KMAX = max(K0, K1)
TCH = NS * (K0 + K1)
E_PAD = TCH * CH
DCPT = TCH // NW


def _make_sc_scatter(H):
    mesh = plsc.VectorSubcoreMesh(core_axis_name="c", subcore_axis_name="s")

    @functools.partial(
        pl.kernel,
        out_type=jax.ShapeDtypeStruct((NC, NPAD, H), jnp.float32),
        mesh=mesh,
        scratch_types=[
            pltpu.VMEM((KMAX, CH), jnp.int32),
            pltpu.VMEM((KMAX, CH), jnp.int32),
            [pltpu.VMEM((CH, H), jnp.float32)] * NB,
            pltpu.VMEM_SHARED((NPAD, H), jnp.float32),
            pltpu.VMEM_SHARED((NPAD, H), jnp.float32),
            [pltpu.SemaphoreType.DMA] * NB,
        ],
        compiler_params=pltpu.CompilerParams(use_tc_tiling_on_sc=False),
    )
    def k(y_hbm, src_hbm, dst_hbm, out_hbm, src_i, dst_i, rows, acc_sh,
          y_sh, gsems):
        cid = lax.axis_index("c")
        sid = lax.axis_index("s")

        pltpu.sync_copy(y_hbm.at[pl.ds(sid * RPT, RPT)],
                        y_sh.at[pl.ds(sid * RPT, RPT)])

        zero16 = jnp.zeros((16,), jnp.float32)

        def zrow(r, carry):
            for j in range(H // 16):
                rows[0][r, pl.ds(j * 16, 16)] = zero16
            return carry

        lax.fori_loop(0, CH, zrow, 0)
        for t in range(RPT // CH):
            pltpu.sync_copy(rows[0], acc_sh.at[pl.ds(sid * RPT + t * CH, CH)])
        plsc.subcore_barrier()

        def run(base_row, K):
            if K == 0:
                return
            pltpu.sync_copy(src_hbm.at[pl.ds(base_row, K)],
                            src_i.at[pl.ds(0, K)])
            pltpu.sync_copy(dst_hbm.at[pl.ds(base_row, K)],
                            dst_i.at[pl.ds(0, K)])
            for b in range(NB):
                pltpu.async_copy(y_sh.at[src_i.at[b]], rows[b], gsems[b])

            def body(o, carry):
                for b in range(NB):
                    g = o * NB + b
                    pltpu.make_async_copy(y_sh.at[src_i.at[g]], rows[b],
                                          gsems[b]).wait()
                    pltpu.sync_copy(rows[b], acc_sh.at[dst_i.at[g]], add=True)

                    @pl.when(g + NB < K)
                    def _():
                        pltpu.async_copy(y_sh.at[src_i.at[g + NB]], rows[b],
                                         gsems[b])
                return carry

            lax.fori_loop(0, K // NB, body, 0)

        @pl.when(cid == 0)
        def _():
            run(sid * K0, K0)

        @pl.when(cid == 1)
        def _():
            run(NS * K0 + sid * K1, K1)

        plsc.subcore_barrier()
        pltpu.sync_copy(
            acc_sh.at[pl.ds(sid * RPT, RPT)],
            out_hbm.at[cid, pl.ds(sid * RPT, RPT)],
        )

    return k


def _sc_degree():
    mesh = plsc.VectorSubcoreMesh(core_axis_name="c", subcore_axis_name="s")

    @functools.partial(
        pl.kernel,
        out_type=jax.ShapeDtypeStruct((NC, NPAD, 16), jnp.float32),
        mesh=mesh,
        scratch_types=[
            pltpu.VMEM((DCPT, CH), jnp.int32),
            pltpu.VMEM((CH, 16), jnp.float32),
            pltpu.VMEM_SHARED((NPAD, 16), jnp.float32),
        ],
        compiler_params=pltpu.CompilerParams(use_tc_tiling_on_sc=False),
    )
    def k(dst_hbm, out_hbm, dst_i, ones_v, acc_sh):
        cid = lax.axis_index("c")
        sid = lax.axis_index("s")
        base_row = (cid * NS + sid) * DCPT

        pltpu.sync_copy(dst_hbm.at[pl.ds(base_row, DCPT)], dst_i)

        zero16 = jnp.zeros((16,), jnp.float32)
        one16 = jnp.ones((16,), jnp.float32)

        def frow(r, carry):
            ones_v[r, :] = zero16
            return carry

        lax.fori_loop(0, CH, frow, 0)
        for t in range(RPT // CH):
            pltpu.sync_copy(ones_v, acc_sh.at[pl.ds(sid * RPT + t * CH, CH)])

        def orow(r, carry):
            ones_v[r, :] = one16
            return carry

        lax.fori_loop(0, CH, orow, 0)
        plsc.subcore_barrier()

        def body(g, carry):
            pltpu.sync_copy(ones_v, acc_sh.at[dst_i.at[g]], add=True)
            return carry

        lax.fori_loop(0, DCPT, body, 0)
        plsc.subcore_barrier()
        pltpu.sync_copy(
            acc_sh.at[pl.ds(sid * RPT, RPT)],
            out_hbm.at[cid, pl.ds(sid * RPT, RPT)],
        )

    return k


_sc_scatter64 = _make_sc_scatter(64)
_sc_scatter32 = _make_sc_scatter(32)
_sc_deg = _sc_degree()

BR = 2048


def _prep1_body(x_ref, w_ref, degp_ref, y_ref, dinv_ref):
    i = pl.program_id(0)
    deg = degp_ref[0, :, 0:1] + degp_ref[1, :, 0:1] + 1.0
    rows = lax.broadcasted_iota(jnp.int32, (BR, 1), 0) + i * BR
    dinv = jnp.where(rows < N, lax.rsqrt(deg), 0.0)
    dinv_ref[...] = dinv
    y_ref[...] = dinv * jnp.dot(x_ref[...], w_ref[...],
                                preferred_element_type=jnp.float32)


def _comb_body(sp_ref, y_ref, dinv_ref, b_ref, w_ref, ynext_ref):
    dinv = dinv_ref[...]
    h = jax.nn.relu(dinv * (sp_ref[0] + sp_ref[1] + y_ref[...]) + b_ref[...])
    ynext_ref[...] = dinv * jnp.dot(h, w_ref[...],
                                    preferred_element_type=jnp.float32)


def _final_body(sp_ref, y_ref, dinv_ref, b_ref, batch_ref, wf1_ref, bf1_ref,
                wf2_ref, bf2_ref, out_ref):
    dinv = dinv_ref[...]
    h = jax.nn.relu(dinv * (sp_ref[0] + sp_ref[1] + y_ref[...]) + b_ref[...])
    gids = lax.broadcasted_iota(jnp.int32, (NG, NPAD), 0)
    m = (batch_ref[...] == gids).astype(jnp.float32)
    sums = jnp.dot(m, h, preferred_element_type=jnp.float32)
    counts = jnp.sum(m, axis=1, keepdims=True)
    pooled = sums / jnp.maximum(counts, 1.0)
    h2 = jax.nn.relu(jnp.dot(pooled, wf1_ref[...],
                             preferred_element_type=jnp.float32) + bf1_ref[...])
    out_ref[...] = jnp.dot(h2, wf2_ref[...],
                           preferred_element_type=jnp.float32) + bf2_ref[...]


def _prep1(x_pad, W1, degp):
    grid = (NPAD // BR,)
    return pl.pallas_call(
        _prep1_body,
        grid=grid,
        in_specs=[
            pl.BlockSpec((BR, F), lambda i: (i, 0)),
            pl.BlockSpec((F, 64), lambda i: (0, 0)),
            pl.BlockSpec((NC, BR, 16), lambda i: (0, i, 0)),
        ],
        out_specs=[
            pl.BlockSpec((BR, 64), lambda i: (i, 0)),
            pl.BlockSpec((BR, 1), lambda i: (i, 0)),
        ],
        out_shape=[
            jax.ShapeDtypeStruct((NPAD, 64), jnp.float32),
            jax.ShapeDtypeStruct((NPAD, 1), jnp.float32),
        ],
    )(x_pad, W1, degp)


def _comb(sp, y, dinv, b, W, Hin, Hout):
    grid = (NPAD // BR,)
    return pl.pallas_call(
        _comb_body,
        grid=grid,
        in_specs=[
            pl.BlockSpec((NC, BR, Hin), lambda i: (0, i, 0)),
            pl.BlockSpec((BR, Hin), lambda i: (i, 0)),
            pl.BlockSpec((BR, 1), lambda i: (i, 0)),
            pl.BlockSpec((1, Hin), lambda i: (0, 0)),
            pl.BlockSpec((Hin, Hout), lambda i: (0, 0)),
        ],
        out_specs=pl.BlockSpec((BR, Hout), lambda i: (i, 0)),
        out_shape=jax.ShapeDtypeStruct((NPAD, Hout), jnp.float32),
    )(sp, y, dinv, b, W)


def _final(sp, y, dinv, b3, batch_row, Wf1, bf1, Wf2, bf2):
    return pl.pallas_call(
        _final_body,
        out_shape=jax.ShapeDtypeStruct((NG, 10), jnp.float32),
    )(sp, y, dinv, b3, batch_row, Wf1, bf1, Wf2, bf2)


def kernel(x, edge_index, batch, W1, b1, W2, b2, W3, b3, Wf1, bf1, Wf2, bf2):
    src = jnp.concatenate([edge_index[0],
                           jnp.full((E_PAD - E,), N, jnp.int32)])
    dst = jnp.concatenate([edge_index[1],
                           jnp.full((E_PAD - E,), N, jnp.int32)])
    src = src.reshape(TCH, CH)
    dst = dst.reshape(TCH, CH)
    x_pad = jnp.zeros((NPAD, F), jnp.float32).at[:N].set(x)
    batch_row = jnp.full((1, NPAD), NG, jnp.int32).at[0, :N].set(batch)

    degp = _sc_deg(dst)

    y1, dinv = _prep1(x_pad, W1, degp)
    s1 = _sc_scatter64(y1, src, dst)
    y2 = _comb(s1, y1, dinv, b1.reshape(1, 64), W2, 64, 64)
    s2 = _sc_scatter64(y2, src, dst)
    y3 = _comb(s2, y2, dinv, b2.reshape(1, 64), W3, 64, 32)
    s3 = _sc_scatter32(y3, src, dst)
    return _final(s3, y3, dinv, b3.reshape(1, 32), batch_row,
                  Wf1, bf1.reshape(1, 32), Wf2, bf2.reshape(1, 10))


# speedup vs baseline: 1.8988x; 1.0057x over previous
import functools

import jax
import jax.numpy as jnp
from jax import lax
from jax.experimental import pallas as pl
from jax.experimental.pallas import tpu as pltpu
from jax.experimental.pallas import tpu_sc as plsc

N = 10000
E = 320000
F = 128
NG = 64

NC = 2
NS = 16
NW = NC * NS
CH = 128
NPAD = 10240
RPT = NPAD // NS
NB = 2

K0 = 80
K1 = 80
KMAX = max(K0, K1)
TCH = NS * (K0 + K1)
E_PAD = TCH * CH
DCPT = TCH // NW


def _make_sc_scatter(H):
    mesh = plsc.VectorSubcoreMesh(core_axis_name="c", subcore_axis_name="s")

    @functools.partial(
        pl.kernel,
        out_type=jax.ShapeDtypeStruct((NC, NPAD, H), jnp.float32),
        mesh=mesh,
        scratch_types=[
            pltpu.VMEM((KMAX, CH), jnp.int32),
            pltpu.VMEM((KMAX, CH), jnp.int32),
            [pltpu.VMEM((CH, H), jnp.float32)] * NB,
            pltpu.VMEM_SHARED((NPAD, H), jnp.float32),
            pltpu.VMEM_SHARED((NPAD, H), jnp.float32),
            [pltpu.SemaphoreType.DMA] * NB,
        ],
        compiler_params=pltpu.CompilerParams(use_tc_tiling_on_sc=False),
    )
    def k(y_hbm, src_hbm, dst_hbm, out_hbm, src_i, dst_i, rows, acc_sh,
          y_sh, gsems):
        cid = lax.axis_index("c")
        sid = lax.axis_index("s")

        pltpu.sync_copy(y_hbm.at[pl.ds(sid * RPT, RPT)],
                        y_sh.at[pl.ds(sid * RPT, RPT)])

        zero16 = jnp.zeros((16,), jnp.float32)

        def zrow(r, carry):
            for j in range(H // 16):
                rows[0][r, pl.ds(j * 16, 16)] = zero16
            return carry

        lax.fori_loop(0, CH, zrow, 0)
        for t in range(RPT // CH):
            pltpu.sync_copy(rows[0], acc_sh.at[pl.ds(sid * RPT + t * CH, CH)])
        plsc.subcore_barrier()

        def run(base_row, K):
            if K == 0:
                return
            pltpu.sync_copy(src_hbm.at[pl.ds(base_row, K)],
                            src_i.at[pl.ds(0, K)])
            pltpu.sync_copy(dst_hbm.at[pl.ds(base_row, K)],
                            dst_i.at[pl.ds(0, K)])
            for b in range(NB):
                pltpu.async_copy(y_sh.at[src_i.at[b]], rows[b], gsems[b])

            def body(o, carry):
                for b in range(NB):
                    g = o * NB + b
                    pltpu.make_async_copy(y_sh.at[src_i.at[g]], rows[b],
                                          gsems[b]).wait()
                    pltpu.sync_copy(rows[b], acc_sh.at[dst_i.at[g]], add=True)

                    @pl.when(g + NB < K)
                    def _():
                        pltpu.async_copy(y_sh.at[src_i.at[g + NB]], rows[b],
                                         gsems[b])
                return carry

            lax.fori_loop(0, K // NB, body, 0)

        @pl.when(cid == 0)
        def _():
            run(sid * K0, K0)

        @pl.when(cid == 1)
        def _():
            run(NS * K0 + sid * K1, K1)

        plsc.subcore_barrier()
        pltpu.sync_copy(
            acc_sh.at[pl.ds(sid * RPT, RPT)],
            out_hbm.at[cid, pl.ds(sid * RPT, RPT)],
        )

    return k


def _sc_degree():
    mesh = plsc.VectorSubcoreMesh(core_axis_name="c", subcore_axis_name="s")

    @functools.partial(
        pl.kernel,
        out_type=jax.ShapeDtypeStruct((NC, NPAD, 16), jnp.float32),
        mesh=mesh,
        scratch_types=[
            pltpu.VMEM((DCPT, CH), jnp.int32),
            pltpu.VMEM((CH, 16), jnp.float32),
            pltpu.VMEM_SHARED((NPAD, 16), jnp.float32),
        ],
        compiler_params=pltpu.CompilerParams(use_tc_tiling_on_sc=False),
    )
    def k(dst_hbm, out_hbm, dst_i, ones_v, acc_sh):
        cid = lax.axis_index("c")
        sid = lax.axis_index("s")
        base_row = (cid * NS + sid) * DCPT

        pltpu.sync_copy(dst_hbm.at[pl.ds(base_row, DCPT)], dst_i)

        zero16 = jnp.zeros((16,), jnp.float32)
        one16 = jnp.ones((16,), jnp.float32)

        def frow(r, carry):
            ones_v[r, :] = zero16
            return carry

        lax.fori_loop(0, CH, frow, 0)
        for t in range(RPT // CH):
            pltpu.sync_copy(ones_v, acc_sh.at[pl.ds(sid * RPT + t * CH, CH)])

        def orow(r, carry):
            ones_v[r, :] = one16
            return carry

        lax.fori_loop(0, CH, orow, 0)
        plsc.subcore_barrier()

        def body(g, carry):
            pltpu.sync_copy(ones_v, acc_sh.at[dst_i.at[g]], add=True)
            return carry

        lax.fori_loop(0, DCPT, body, 0)
        plsc.subcore_barrier()
        pltpu.sync_copy(
            acc_sh.at[pl.ds(sid * RPT, RPT)],
            out_hbm.at[cid, pl.ds(sid * RPT, RPT)],
        )

    return k


_sc_scatter64 = _make_sc_scatter(64)
_sc_scatter32 = _make_sc_scatter(32)
_sc_deg = _sc_degree()

BR = 5120


def _prep1_body(x_ref, w_ref, degp_ref, y_ref, dinv_ref):
    i = pl.program_id(0)
    deg = degp_ref[0, :, 0:1] + degp_ref[1, :, 0:1] + 1.0
    rows = lax.broadcasted_iota(jnp.int32, (BR, 1), 0) + i * BR
    dinv = jnp.where(rows < N, lax.rsqrt(deg), 0.0)
    dinv_ref[...] = dinv
    y_ref[...] = dinv * jnp.dot(x_ref[...], w_ref[...],
                                preferred_element_type=jnp.float32)


def _comb_body(sp_ref, y_ref, dinv_ref, b_ref, w_ref, ynext_ref):
    dinv = dinv_ref[...]
    h = jax.nn.relu(dinv * (sp_ref[0] + sp_ref[1] + y_ref[...]) + b_ref[...])
    ynext_ref[...] = dinv * jnp.dot(h, w_ref[...],
                                    preferred_element_type=jnp.float32)


def _final_body(sp_ref, y_ref, dinv_ref, b_ref, batch_ref, wf1_ref, bf1_ref,
                wf2_ref, bf2_ref, out_ref):
    dinv = dinv_ref[...]
    h = jax.nn.relu(dinv * (sp_ref[0] + sp_ref[1] + y_ref[...]) + b_ref[...])
    gids = lax.broadcasted_iota(jnp.int32, (NG, NPAD), 0)
    m = (batch_ref[...] == gids).astype(jnp.float32)
    sums = jnp.dot(m, h, preferred_element_type=jnp.float32)
    counts = jnp.sum(m, axis=1, keepdims=True)
    pooled = sums / jnp.maximum(counts, 1.0)
    h2 = jax.nn.relu(jnp.dot(pooled, wf1_ref[...],
                             preferred_element_type=jnp.float32) + bf1_ref[...])
    out_ref[...] = jnp.dot(h2, wf2_ref[...],
                           preferred_element_type=jnp.float32) + bf2_ref[...]


def _prep1(x_pad, W1, degp):
    grid = (NPAD // BR,)
    return pl.pallas_call(
        _prep1_body,
        grid=grid,
        in_specs=[
            pl.BlockSpec((BR, F), lambda i: (i, 0)),
            pl.BlockSpec((F, 64), lambda i: (0, 0)),
            pl.BlockSpec((NC, BR, 16), lambda i: (0, i, 0)),
        ],
        out_specs=[
            pl.BlockSpec((BR, 64), lambda i: (i, 0)),
            pl.BlockSpec((BR, 1), lambda i: (i, 0)),
        ],
        out_shape=[
            jax.ShapeDtypeStruct((NPAD, 64), jnp.float32),
            jax.ShapeDtypeStruct((NPAD, 1), jnp.float32),
        ],
    )(x_pad, W1, degp)


def _comb(sp, y, dinv, b, W, Hin, Hout):
    grid = (NPAD // BR,)
    return pl.pallas_call(
        _comb_body,
        grid=grid,
        in_specs=[
            pl.BlockSpec((NC, BR, Hin), lambda i: (0, i, 0)),
            pl.BlockSpec((BR, Hin), lambda i: (i, 0)),
            pl.BlockSpec((BR, 1), lambda i: (i, 0)),
            pl.BlockSpec((1, Hin), lambda i: (0, 0)),
            pl.BlockSpec((Hin, Hout), lambda i: (0, 0)),
        ],
        out_specs=pl.BlockSpec((BR, Hout), lambda i: (i, 0)),
        out_shape=jax.ShapeDtypeStruct((NPAD, Hout), jnp.float32),
    )(sp, y, dinv, b, W)


def _final(sp, y, dinv, b3, batch_row, Wf1, bf1, Wf2, bf2):
    return pl.pallas_call(
        _final_body,
        out_shape=jax.ShapeDtypeStruct((NG, 10), jnp.float32),
    )(sp, y, dinv, b3, batch_row, Wf1, bf1, Wf2, bf2)


def kernel(x, edge_index, batch, W1, b1, W2, b2, W3, b3, Wf1, bf1, Wf2, bf2):
    src = jnp.concatenate([edge_index[0],
                           jnp.full((E_PAD - E,), N, jnp.int32)])
    dst = jnp.concatenate([edge_index[1],
                           jnp.full((E_PAD - E,), N, jnp.int32)])
    src = src.reshape(TCH, CH)
    dst = dst.reshape(TCH, CH)
    x_pad = jnp.zeros((NPAD, F), jnp.float32).at[:N].set(x)
    batch_row = jnp.full((1, NPAD), NG, jnp.int32).at[0, :N].set(batch)

    degp = _sc_deg(dst)

    y1, dinv = _prep1(x_pad, W1, degp)
    s1 = _sc_scatter64(y1, src, dst)
    y2 = _comb(s1, y1, dinv, b1.reshape(1, 64), W2, 64, 64)
    s2 = _sc_scatter64(y2, src, dst)
    y3 = _comb(s2, y2, dinv, b2.reshape(1, 64), W3, 64, 32)
    s3 = _sc_scatter32(y3, src, dst)
    return _final(s3, y3, dinv, b3.reshape(1, 32), batch_row,
                  Wf1, bf1.reshape(1, 32), Wf2, bf2.reshape(1, 10))
